# baseline (speedup 1.0000x reference)
import functools

import jax
import jax.numpy as jnp
from jax import lax
from jax.experimental import pallas as pl
from jax.experimental.pallas import tpu as pltpu
from jax.experimental.pallas import tpu_sc as plsc

EMB_DIM = 64
NUM_CORES = 2
NUM_SUBCORES = 16
NUM_WORKERS = NUM_CORES * NUM_SUBCORES
CHUNK = 128


@jax.jit
def _sc_gather(idx_flat, table):
    B = idx_flat.shape[0]
    b_per_w = B // NUM_WORKERS
    n_ch = b_per_w // CHUNK
    mesh = plsc.VectorSubcoreMesh(core_axis_name="c", subcore_axis_name="s")

    @functools.partial(
        pl.kernel,
        mesh=mesh,
        out_type=jax.ShapeDtypeStruct((B, EMB_DIM), jnp.float32),
        compiler_params=pltpu.CompilerParams(use_tc_tiling_on_sc=False),
        scratch_types=[
            pltpu.VMEM((CHUNK,), jnp.int32),
            pltpu.VMEM((CHUNK, EMB_DIM), jnp.float32),
            pltpu.SemaphoreType.DMA,
        ],
    )
    def k(table_hbm, idx_hbm, out_hbm, idx_v, rows_v, sem):
        wid = lax.axis_index("s") * NUM_CORES + lax.axis_index("c")
        base = wid * b_per_w

        @pl.loop(0, n_ch)
        def _(i):
            off = base + i * CHUNK
            pltpu.sync_copy(idx_hbm.at[pl.ds(off, CHUNK)], idx_v)
            pltpu.async_copy(table_hbm.at[idx_v], rows_v, sem).wait()
            pltpu.sync_copy(rows_v, out_hbm.at[pl.ds(off, CHUNK)])

    return k(table, idx_flat)


def kernel(x, table):
    B = x.shape[0] * x.shape[1]
    out = _sc_gather(x.reshape(B), table)
    return out.reshape(x.shape[0], x.shape[1], EMB_DIM)


# speedup vs baseline: 1.0783x; 1.0783x over previous
import functools

import jax
import jax.numpy as jnp
from jax import lax
from jax.experimental import pallas as pl
from jax.experimental.pallas import tpu as pltpu
from jax.experimental.pallas import tpu_sc as plsc

EMB_DIM = 64
NUM_CORES = 2
NUM_SUBCORES = 16
NUM_WORKERS = NUM_CORES * NUM_SUBCORES
CHUNK = 128
GROUP = 5
NGROUPS = 10


@jax.jit
def _sc_gather(idx3, table):
    n_ch = GROUP * NGROUPS
    b_per_w = n_ch * CHUNK
    B = NUM_WORKERS * b_per_w
    grp_rows = GROUP * CHUNK
    mesh = plsc.VectorSubcoreMesh(core_axis_name="c", subcore_axis_name="s")

    @functools.partial(
        pl.kernel,
        mesh=mesh,
        out_type=jax.ShapeDtypeStruct((B, EMB_DIM), jnp.float32),
        compiler_params=pltpu.CompilerParams(use_tc_tiling_on_sc=False),
        scratch_types=[
            pltpu.VMEM((n_ch, CHUNK), jnp.int32),
            pltpu.VMEM((grp_rows, EMB_DIM), jnp.float32),
            pltpu.VMEM((grp_rows, EMB_DIM), jnp.float32),
            pltpu.SemaphoreType.DMA,
            pltpu.SemaphoreType.DMA,
        ],
    )
    def k(table_hbm, idx_hbm, out_hbm, idx_v, buf_a, buf_b, sem_a, sem_b):
        wid = lax.axis_index("s") * NUM_CORES + lax.axis_index("c")
        base = wid * b_per_w
        pltpu.sync_copy(idx_hbm.at[wid], idx_v)

        def fire(g, buf, sem):
            for j in range(GROUP):
                pltpu.make_async_copy(
                    table_hbm.at[idx_v.at[g * GROUP + j]],
                    buf.at[pl.ds(j * CHUNK, CHUNK)],
                    sem,
                ).start()

        def drain(g, buf, sem):
            for j in range(GROUP):
                pltpu.make_async_copy(
                    table_hbm.at[idx_v.at[g * GROUP + j]],
                    buf.at[pl.ds(j * CHUNK, CHUNK)],
                    sem,
                ).wait()
            pltpu.sync_copy(buf, out_hbm.at[pl.ds(base + g * grp_rows, grp_rows)])

        fire(0, buf_a, sem_a)

        @pl.loop(0, NGROUPS, step=2)
        def _(g):
            @pl.when(g + 1 < NGROUPS)
            def _():
                fire(g + 1, buf_b, sem_b)

            drain(g, buf_a, sem_a)

            @pl.when(g + 2 < NGROUPS)
            def _():
                fire(g + 2, buf_a, sem_a)

            @pl.when(g + 1 < NGROUPS)
            def _():
                drain(g + 1, buf_b, sem_b)

    return k(table, idx3)


def kernel(x, table):
    B = x.shape[0] * x.shape[1]
    n_ch = GROUP * NGROUPS
    idx3 = x.reshape(NUM_WORKERS, n_ch, CHUNK)
    out = _sc_gather(idx3, table)
    return out.reshape(x.shape[0], x.shape[1], EMB_DIM)
